# baseline (speedup 1.0000x reference)
import jax
import jax.numpy as jnp
from jax.experimental import pallas as pl
from jax.experimental.pallas import tpu as pltpu

N_ROWS = 20000
BATCH = 1024
FDIM = 64
LDIM = 32
LPAD = 128
ETA_C = 0.01
TILE_N = 2000
NT = N_ROWS // TILE_N
F32 = jnp.float32
BF16 = jnp.bfloat16


def _fused(fs_ref, f1_ref, f2_ref, ls_ref, li1_ref, li2_ref, xt2_ref,
           w1_ref, w2_ref, b1_ref, b2_ref, u1_ref, u2_ref, ld1_ref, ld2_ref,
           y_ref, s12_s, z_s, p1_s, p2_s, a1_s, a2_s):
    i = pl.program_id(0)

    @pl.when(i < NT)
    def _phase_a():
        fs = fs_ref[...]
        fssq = jnp.sum(fs * fs, axis=1, keepdims=True)
        arg = jnp.dot(fs.astype(BF16), xt2_ref[...],
                      preferred_element_type=F32)
        e = jnp.exp(arg)
        w = jnp.exp(-fssq)
        f12w = jnp.concatenate([f1_ref[...], f2_ref[...]], axis=1) * w
        s12 = jax.lax.dot_general(f12w, e, (((0,), (0,)), ((), ())),
                                  preferred_element_type=F32)
        zc = jax.lax.dot_general(w, e, (((0,), (0,)), ((), ())),
                                 preferred_element_type=F32)

        @pl.when(i == 0)
        def _():
            s12_s[...] = s12
            z_s[...] = zc

        @pl.when(i > 0)
        def _():
            s12_s[...] += s12
            z_s[...] += zc

    @pl.when(i == NT)
    def _phase_b():
        z = z_s[...]
        lio = jax.lax.broadcasted_iota(jnp.int32, (LPAD, BATCH), 0)

        def side(s, w, b, u, ldm, p_out):
            xtt = s / z
            m = jax.lax.dot_general(w, xtt, (((0,), (0,)), ((), ())),
                                    preferred_element_type=F32)
            yt = m + b
            cross = jnp.dot(u, yt, preferred_element_type=F32)
            usq = jnp.sum(u * u, axis=1, keepdims=True)
            ysq = jnp.sum(yt * yt, axis=0, keepdims=True)
            score = jnp.maximum(usq - 2.0 * cross + ysq, 0.0)
            mn = jnp.min(score, axis=0, keepdims=True)
            idx = jnp.min(jnp.where(score == mn, lio, LPAD),
                          axis=0, keepdims=True)
            oh = (lio == idx).astype(F32)
            g = jnp.dot(ldm, oh, preferred_element_type=F32)
            p_out[0:FDIM, :] = (2.0 * xtt).astype(BF16)
            p_out[FDIM:FDIM + LPAD, :] = (-ETA_C * g).astype(BF16)

        s12 = s12_s[...]
        side(s12[0:FDIM, :], w1_ref[...], b1_ref[...], u1_ref[...],
             ld1_ref[...], p1_s)
        side(s12[FDIM:2 * FDIM, :], w2_ref[...], b2_ref[...], u2_ref[...],
             ld2_ref[...], p2_s)

    @pl.when(i > NT)
    def _phase_c():
        lio = jax.lax.broadcasted_iota(jnp.int32, (TILE_N, LPAD), 1)
        ls = ls_ref[...]

        def side(f_ref, li_ref, p_s):
            f = f_ref[...]
            fsq = jnp.sum(f * f, axis=1, keepdims=True)
            oh = (li_ref[...] == lio).astype(BF16)
            arg = jnp.dot(jnp.concatenate([f.astype(BF16), oh], axis=1),
                          p_s[...], preferred_element_type=F32)
            e = jnp.exp(arg)
            w = jnp.exp(-fsq)
            lsw = jnp.concatenate([ls * w, w], axis=1)
            return jax.lax.dot_general(lsw, e, (((0,), (0,)), ((), ())),
                                       preferred_element_type=F32)

        ta = side(f1_ref, li1_ref, p1_s)
        tb = side(f2_ref, li2_ref, p2_s)

        @pl.when(i == NT + 1)
        def _():
            a1_s[...] = ta
            a2_s[...] = tb

        @pl.when(i > NT + 1)
        def _():
            a1_s[...] += ta
            a2_s[...] += tb

        @pl.when(i == 2 * NT)
        def _():
            a1 = a1_s[...]
            a2 = a2_s[...]
            y_ref[...] = jnp.transpose(
                0.5 * (a1[0:LDIM, :] / a1[LDIM:LDIM + 1, :]
                       + a2[0:LDIM, :] / a2[LDIM:LDIM + 1, :]))


def _a_idx(i):
    return (jnp.minimum(i, NT - 1), 0)


def _ac_idx(i):
    return (jnp.where(i < NT, i, jnp.clip(i - NT - 1, 0, NT - 1)), 0)


def _c_idx(i):
    return (jnp.clip(i - NT - 1, 0, NT - 1), 0)


def _const_idx(i):
    return (0, 0)


@jax.jit
def _impl(x, features_star, labels_star, features_1, features_2,
          unique_labels_1, unique_labels_2, label_indices_1, label_indices_2,
          label_distances_1, label_distances_2, W1, b1, W2, b2):
    xt2 = (2.0 * x.T).astype(BF16)
    u1p = jnp.pad(unique_labels_1, ((0, LPAD - 100), (0, 0)),
                  constant_values=1e6)
    u2p = jnp.pad(unique_labels_2, ((0, LPAD - 100), (0, 0)),
                  constant_values=1e6)
    ld1p = jnp.pad(label_distances_1, ((0, LPAD - 100), (0, LPAD - 100)))
    ld2p = jnp.pad(label_distances_2, ((0, LPAD - 100), (0, LPAD - 100)))
    li1c = label_indices_1.astype(jnp.int32).reshape(N_ROWS, 1)
    li2c = label_indices_2.astype(jnp.int32).reshape(N_ROWS, 1)
    b1c = b1.reshape(LDIM, 1)
    b2c = b2.reshape(LDIM, 1)

    y = pl.pallas_call(
        _fused,
        grid=(2 * NT + 1,),
        in_specs=[
            pl.BlockSpec((TILE_N, FDIM), _a_idx),
            pl.BlockSpec((TILE_N, FDIM), _ac_idx),
            pl.BlockSpec((TILE_N, FDIM), _ac_idx),
            pl.BlockSpec((TILE_N, LDIM), _c_idx),
            pl.BlockSpec((TILE_N, 1), _c_idx),
            pl.BlockSpec((TILE_N, 1), _c_idx),
            pl.BlockSpec((FDIM, BATCH), _const_idx),
            pl.BlockSpec((FDIM, LDIM), _const_idx),
            pl.BlockSpec((FDIM, LDIM), _const_idx),
            pl.BlockSpec((LDIM, 1), _const_idx),
            pl.BlockSpec((LDIM, 1), _const_idx),
            pl.BlockSpec((LPAD, LDIM), _const_idx),
            pl.BlockSpec((LPAD, LDIM), _const_idx),
            pl.BlockSpec((LPAD, LPAD), _const_idx),
            pl.BlockSpec((LPAD, LPAD), _const_idx),
        ],
        out_specs=pl.BlockSpec((BATCH, LDIM), _const_idx),
        out_shape=jax.ShapeDtypeStruct((BATCH, LDIM), F32),
        scratch_shapes=[pltpu.VMEM((2 * FDIM, BATCH), F32),
                        pltpu.VMEM((1, BATCH), F32),
                        pltpu.VMEM((FDIM + LPAD, BATCH), BF16),
                        pltpu.VMEM((FDIM + LPAD, BATCH), BF16),
                        pltpu.VMEM((LDIM + 1, BATCH), F32),
                        pltpu.VMEM((LDIM + 1, BATCH), F32)],
        compiler_params=pltpu.CompilerParams(
            dimension_semantics=("arbitrary",)),
    )(features_star, features_1, features_2, labels_star, li1c, li2c, xt2,
      W1, W2, b1c, b2c, u1p, u2p, ld1p, ld2p)
    return y


def kernel(x, features_star, labels_star, features_1, features_2,
           unique_labels_1, unique_labels_2, label_indices_1, label_indices_2,
           label_distances_1, label_distances_2, W1, b1, W2, b2):
    return _impl(x, features_star, labels_star, features_1, features_2,
                 unique_labels_1, unique_labels_2, label_indices_1,
                 label_indices_2, label_distances_1, label_distances_2,
                 W1, b1, W2, b2)


# speedup vs baseline: 1.1200x; 1.1200x over previous
import jax
import jax.numpy as jnp
from jax.experimental import pallas as pl
from jax.experimental.pallas import tpu as pltpu

N_ROWS = 20000
BATCH = 1024
FDIM = 64
LDIM = 32
NLAB = 100
ETA_C = 0.01
TILE_N = 2000
NT = N_ROWS // TILE_N
F32 = jnp.float32


def _fused(fs_ref, f1_ref, f2_ref, ls_ref, li1_ref, li2_ref, x_ref,
           w1_ref, w2_ref, b1_ref, b2_ref, u1_ref, u2_ref, ld1_ref, ld2_ref,
           y_ref, xt2_s, s12z_s, p1_s, p2_s, a1_s, a2_s):
    i = pl.program_id(0)

    @pl.when(i == 0)
    def _prep():
        xt2_s[...] = 2.0 * jnp.transpose(x_ref[...])

    @pl.when(i < NT)
    def _phase_a():
        fs = fs_ref[...]
        fssq = jnp.sum(fs * fs, axis=1, keepdims=True)
        arg = jnp.dot(fs, xt2_s[...], preferred_element_type=F32)
        e = jnp.exp(arg)
        w = jnp.exp(-fssq)
        f12w = jnp.concatenate(
            [f1_ref[...] * w, f2_ref[...] * w, w], axis=1)
        s12z = jax.lax.dot_general(f12w, e, (((0,), (0,)), ((), ())),
                                   preferred_element_type=F32)

        @pl.when(i == 0)
        def _():
            s12z_s[...] = s12z

        @pl.when(i > 0)
        def _():
            s12z_s[...] += s12z

    @pl.when(i == NT)
    def _phase_b():
        s12z = s12z_s[...]
        z = s12z[2 * FDIM:2 * FDIM + 1, :]
        lio = jax.lax.broadcasted_iota(jnp.int32, (NLAB, BATCH), 0)

        def side(s, w, b, u, ldm, p_out):
            xtt = s / z
            m = jax.lax.dot_general(w, xtt, (((0,), (0,)), ((), ())),
                                    preferred_element_type=F32)
            yt = m + b
            cross = jnp.dot(u, yt, preferred_element_type=F32)
            usq = jnp.sum(u * u, axis=1, keepdims=True)
            ysq = jnp.sum(yt * yt, axis=0, keepdims=True)
            score = jnp.maximum(usq - 2.0 * cross + ysq, 0.0)
            mn = jnp.min(score, axis=0, keepdims=True)
            idx = jnp.min(jnp.where(score == mn, lio, NLAB),
                          axis=0, keepdims=True)
            oh = (lio == idx).astype(F32)
            g = jnp.dot(ldm, oh, preferred_element_type=F32)
            p_out[0:FDIM, :] = 2.0 * xtt
            p_out[FDIM:FDIM + NLAB, :] = -ETA_C * g

        side(s12z[0:FDIM, :], w1_ref[...], b1_ref[...], u1_ref[...],
             ld1_ref[...], p1_s)
        side(s12z[FDIM:2 * FDIM, :], w2_ref[...], b2_ref[...], u2_ref[...],
             ld2_ref[...], p2_s)

    @pl.when(i > NT)
    def _phase_c():
        lio = jax.lax.broadcasted_iota(jnp.int32, (TILE_N, NLAB), 1)
        ls = ls_ref[...]

        def side(f_ref, li_ref, p_s):
            f = f_ref[...]
            fsq = jnp.sum(f * f, axis=1, keepdims=True)
            oh = (li_ref[...] == lio).astype(F32)
            arg = jnp.dot(jnp.concatenate([f, oh], axis=1), p_s[...],
                          preferred_element_type=F32)
            e = jnp.exp(arg)
            w = jnp.exp(-fsq)
            lsw = jnp.concatenate([ls * w, w], axis=1)
            return jax.lax.dot_general(lsw, e, (((0,), (0,)), ((), ())),
                                       preferred_element_type=F32)

        ta = side(f1_ref, li1_ref, p1_s)
        tb = side(f2_ref, li2_ref, p2_s)

        @pl.when(i == NT + 1)
        def _():
            a1_s[...] = ta
            a2_s[...] = tb

        @pl.when(i > NT + 1)
        def _():
            a1_s[...] += ta
            a2_s[...] += tb

        @pl.when(i == 2 * NT)
        def _():
            a1 = a1_s[...]
            a2 = a2_s[...]
            y_ref[...] = jnp.transpose(
                0.5 * (a1[0:LDIM, :] / a1[LDIM:LDIM + 1, :]
                       + a2[0:LDIM, :] / a2[LDIM:LDIM + 1, :]))


def _a_idx(i):
    return (jnp.minimum(i, NT - 1), 0)


def _ac_idx(i):
    return (jnp.where(i < NT, i, jnp.clip(i - NT - 1, 0, NT - 1)), 0)


def _c_idx(i):
    return (jnp.clip(i - NT - 1, 0, NT - 1), 0)


def _const_idx(i):
    return (0, 0)


@jax.jit
def _impl(x, features_star, labels_star, features_1, features_2,
          unique_labels_1, unique_labels_2, label_indices_1, label_indices_2,
          label_distances_1, label_distances_2, W1, b1, W2, b2):
    li1c = label_indices_1.astype(jnp.int32).reshape(N_ROWS, 1)
    li2c = label_indices_2.astype(jnp.int32).reshape(N_ROWS, 1)
    b1c = b1.reshape(LDIM, 1)
    b2c = b2.reshape(LDIM, 1)

    y = pl.pallas_call(
        _fused,
        grid=(2 * NT + 1,),
        in_specs=[
            pl.BlockSpec((TILE_N, FDIM), _a_idx),
            pl.BlockSpec((TILE_N, FDIM), _ac_idx),
            pl.BlockSpec((TILE_N, FDIM), _ac_idx),
            pl.BlockSpec((TILE_N, LDIM), _c_idx),
            pl.BlockSpec((TILE_N, 1), _c_idx),
            pl.BlockSpec((TILE_N, 1), _c_idx),
            pl.BlockSpec((BATCH, FDIM), _const_idx),
            pl.BlockSpec((FDIM, LDIM), _const_idx),
            pl.BlockSpec((FDIM, LDIM), _const_idx),
            pl.BlockSpec((LDIM, 1), _const_idx),
            pl.BlockSpec((LDIM, 1), _const_idx),
            pl.BlockSpec((NLAB, LDIM), _const_idx),
            pl.BlockSpec((NLAB, LDIM), _const_idx),
            pl.BlockSpec((NLAB, NLAB), _const_idx),
            pl.BlockSpec((NLAB, NLAB), _const_idx),
        ],
        out_specs=pl.BlockSpec((BATCH, LDIM), _const_idx),
        out_shape=jax.ShapeDtypeStruct((BATCH, LDIM), F32),
        scratch_shapes=[pltpu.VMEM((FDIM, BATCH), F32),
                        pltpu.VMEM((2 * FDIM + 1, BATCH), F32),
                        pltpu.VMEM((FDIM + NLAB, BATCH), F32),
                        pltpu.VMEM((FDIM + NLAB, BATCH), F32),
                        pltpu.VMEM((LDIM + 1, BATCH), F32),
                        pltpu.VMEM((LDIM + 1, BATCH), F32)],
        compiler_params=pltpu.CompilerParams(
            dimension_semantics=("arbitrary",)),
    )(features_star, features_1, features_2, labels_star, li1c, li2c, x,
      W1, W2, b1c, b2c, unique_labels_1, unique_labels_2,
      label_distances_1, label_distances_2)
    return y


def kernel(x, features_star, labels_star, features_1, features_2,
           unique_labels_1, unique_labels_2, label_indices_1, label_indices_2,
           label_distances_1, label_distances_2, W1, b1, W2, b2):
    return _impl(x, features_star, labels_star, features_1, features_2,
                 unique_labels_1, unique_labels_2, label_indices_1,
                 label_indices_2, label_distances_1, label_distances_2,
                 W1, b1, W2, b2)
